# baseline (speedup 1.0000x reference)
import functools

import jax
import jax.numpy as jnp
from jax import lax
from jax.experimental import pallas as pl
from jax.experimental.pallas import tpu as pltpu
from jax.experimental.pallas import tpu_sc as plsc


BT = 128


def _moe_block_body(nblk_ref, bexp_ref, xs_ref, sw_ref,
                    w1a_ref, w1b_ref, w2a_ref, w2b_ref, ys_ref,
                    *, d_ff, dh, fh):
    i = pl.program_id(0)

    @pl.when(i < nblk_ref[0])
    def _():
        xb = xs_ref[...]
        gu = (jnp.dot(xb[:, :dh], w1a_ref[0, 0], preferred_element_type=jnp.float32)
              + jnp.dot(xb[:, dh:], w1b_ref[0, 0], preferred_element_type=jnp.float32))
        g = gu[:, :d_ff]
        u = gu[:, d_ff:]
        h = g * jax.nn.sigmoid(g) * u
        yb = (jnp.dot(h[:, :fh], w2a_ref[0, 0], preferred_element_type=jnp.float32)
              + jnp.dot(h[:, fh:], w2b_ref[0, 0], preferred_element_type=jnp.float32))
        ys_ref[...] = yb * sw_ref[...]


def _tc_grouped_matmul(nblocks, bexp, xs, sw, w1, w2, maxb):
    nsp, d = xs.shape
    e, _, f2 = w1.shape
    d_ff = w2.shape[1]
    dh = d // 2
    fh = d_ff // 2
    w1v = w1.reshape(e, 2, dh, f2)
    w2v = w2.reshape(e, 2, fh, d)
    grid_spec = pltpu.PrefetchScalarGridSpec(
        num_scalar_prefetch=2,
        grid=(maxb,),
        in_specs=[
            pl.BlockSpec((BT, d), lambda i, nb, be: (i, 0)),
            pl.BlockSpec((BT, 1), lambda i, nb, be: (i, 0)),
            pl.BlockSpec((1, 1, dh, f2), lambda i, nb, be: (be[i], 0, 0, 0)),
            pl.BlockSpec((1, 1, dh, f2), lambda i, nb, be: (be[i], 1, 0, 0)),
            pl.BlockSpec((1, 1, fh, d), lambda i, nb, be: (be[i], 0, 0, 0)),
            pl.BlockSpec((1, 1, fh, d), lambda i, nb, be: (be[i], 1, 0, 0)),
        ],
        out_specs=pl.BlockSpec((BT, d), lambda i, nb, be: (i, 0)),
    )
    return pl.pallas_call(
        functools.partial(_moe_block_body, d_ff=d_ff, dh=dh, fh=fh),
        grid_spec=grid_spec,
        out_shape=jax.ShapeDtypeStruct((nsp, d), jnp.float32),
        compiler_params=pltpu.CompilerParams(
            dimension_semantics=("arbitrary",),
        ),
    )(nblocks, bexp, xs, sw, w1v, w1v, w2v, w2v)


def _sc_permute_rows(x, tok_sorted, dest, nsp):
    t, d = x.shape
    ns = tok_sorted.shape[0]
    info = plsc.get_sparse_core_info()
    nw = info.num_cores * info.num_subcores
    per_w = ns // nw
    ch = 128
    while per_w % ch:
        ch //= 2
    mesh = plsc.VectorSubcoreMesh(core_axis_name="c", subcore_axis_name="s")

    @functools.partial(
        pl.kernel, mesh=mesh,
        out_type=jax.ShapeDtypeStruct((nsp, d), jnp.float32),
        scratch_types=[
            pltpu.VMEM((ch,), jnp.int32),
            pltpu.VMEM((ch,), jnp.int32),
            pltpu.VMEM((ch, d), jnp.float32),
            pltpu.SemaphoreType.DMA,
        ],
    )
    def k(x_hbm, tok_hbm, dest_hbm, out_hbm, tok_v, dest_v, rows_v, sem):
        wid = lax.axis_index("s") * info.num_cores + lax.axis_index("c")
        base = wid * per_w

        def chunk(c, carry):
            off = base + c * ch
            pltpu.sync_copy(tok_hbm.at[pl.ds(off, ch)], tok_v)
            pltpu.sync_copy(dest_hbm.at[pl.ds(off, ch)], dest_v)
            pltpu.async_copy(x_hbm.at[tok_v], rows_v, sem).wait()
            pltpu.async_copy(rows_v, out_hbm.at[dest_v], sem).wait()
            return carry

        lax.fori_loop(0, per_w // ch, chunk, 0)

    return k(x, tok_sorted, dest)


def _sc_combine(ys, pos_list):
    d = ys.shape[1]
    t = pos_list[0].shape[0]
    info = plsc.get_sparse_core_info()
    nw = info.num_cores * info.num_subcores
    tw = t // nw
    mesh = plsc.VectorSubcoreMesh(core_axis_name="c", subcore_axis_name="s")

    @functools.partial(
        pl.kernel, mesh=mesh,
        out_type=jax.ShapeDtypeStruct((t, d), jnp.float32),
        scratch_types=[
            pltpu.VMEM((tw,), jnp.int32),
            pltpu.VMEM((tw, d), jnp.float32),
            pltpu.VMEM((tw, d), jnp.float32),
            pltpu.SemaphoreType.DMA,
        ],
    )
    def k(ys_hbm, *rest):
        pos_hbms = rest[:len(pos_list)]
        out_hbm, idx_v, acc_v, tmp_v, sem = rest[len(pos_list):]
        wid = lax.axis_index("s") * info.num_cores + lax.axis_index("c")
        base = wid * tw

        pltpu.sync_copy(pos_hbms[0].at[pl.ds(base, tw)], idx_v)
        pltpu.async_copy(ys_hbm.at[idx_v], acc_v, sem).wait()
        for pk in pos_hbms[1:]:
            pltpu.sync_copy(pk.at[pl.ds(base, tw)], idx_v)
            pltpu.async_copy(ys_hbm.at[idx_v], tmp_v, sem).wait()

            def row(r, carry):
                def chunk(c, carry2):
                    sl = pl.ds(c * 16, 16)
                    acc_v[r, sl] = acc_v[r, sl] + tmp_v[r, sl]
                    return carry2
                lax.fori_loop(0, d // 16, chunk, 0)
                return carry

            lax.fori_loop(0, tw, row, 0)
        pltpu.sync_copy(acc_v, out_hbm.at[pl.ds(base, tw)])

    return k(ys, *pos_list)


def _meta_body(ids_ref, dest_ref, meta_ref, *, n_exp, nb, cs_b):
    ids = ids_ref[...]
    e_iota = lax.broadcasted_iota(jnp.int32, (nb, n_exp, cs_b), 1)
    oh = (ids[:, None, :] == e_iota).astype(jnp.float32)
    c1 = lax.broadcasted_iota(jnp.int32, (cs_b, cs_b), 0)
    c2 = lax.broadcasted_iota(jnp.int32, (cs_b, cs_b), 1)
    tri = (c1 <= c2).astype(jnp.float32)
    intra = lax.dot_general(oh, tri, (((2,), (0,)), ((), ())),
                            preferred_element_type=jnp.float32)
    t1 = jnp.sum(oh, axis=2)
    b1 = lax.broadcasted_iota(jnp.int32, (nb, nb), 0)
    b2 = lax.broadcasted_iota(jnp.int32, (nb, nb), 1)
    tri_s = (b2 < b1).astype(jnp.float32)
    boff = jnp.dot(tri_s, t1, preferred_element_type=jnp.float32)
    csum = intra + boff[:, :, None]
    rank = jnp.sum(oh * csum, axis=1) - 1.0
    counts = jnp.sum(t1, axis=0, keepdims=True)
    nblk = jnp.ceil(counts / BT)
    e1 = lax.broadcasted_iota(jnp.int32, (n_exp, n_exp), 0)
    e2 = lax.broadcasted_iota(jnp.int32, (n_exp, n_exp), 1)
    tri_e = (e1 <= e2).astype(jnp.float32)
    incl = jnp.dot(nblk, tri_e, preferred_element_type=jnp.float32)
    blk_off = incl - nblk
    bo_col = jnp.transpose(blk_off)
    bo_slot = jnp.sum(oh * bo_col[None, :, :], axis=1)
    dest_ref[...] = (bo_slot * BT + rank).astype(jnp.int32)
    nblocks = jnp.sum(nblk)
    incl_col = jnp.transpose(incl)
    j_row = lax.broadcasted_iota(jnp.int32, (1, cs_b), 1).astype(jnp.float32)
    bexp = jnp.sum((incl_col <= j_row).astype(jnp.float32), axis=0,
                   keepdims=True)
    e_row = lax.broadcasted_iota(jnp.int32, (1, n_exp), 1).astype(jnp.float32)
    last_e = jnp.max(jnp.where(counts > 0, e_row, 0.0))
    bexp = jnp.where(j_row < nblocks,
                     jnp.minimum(bexp, float(n_exp - 1)), last_e)
    meta = jnp.where(j_row == cs_b - 1, nblocks, bexp).astype(jnp.int32)
    meta_ref[...] = jnp.broadcast_to(meta, (8, cs_b))


def _routing_metadata(topk_weights, topk_ids, n_exp, maxb):
    t, k = topk_ids.shape
    ns = t * k
    nsp = maxb * BT
    cs_b = 128
    nb = ns // cs_b
    dest2, meta = pl.pallas_call(
        functools.partial(_meta_body, n_exp=n_exp, nb=nb, cs_b=cs_b),
        out_shape=(jax.ShapeDtypeStruct((nb, cs_b), jnp.int32),
                   jax.ShapeDtypeStruct((8, cs_b), jnp.int32)),
    )(topk_ids.reshape(nb, cs_b))
    dest = dest2.reshape(ns)
    nblocks = meta[0, cs_b - 1:cs_b]
    bexp = meta[0, :maxb]
    flat_w = topk_weights.reshape(-1)
    tok = jnp.arange(ns, dtype=jnp.int32) // k
    swp = jnp.zeros((nsp,), jnp.float32).at[dest].set(flat_w)
    pos = dest.reshape(t, k)
    pos_list = [pos[:, j] for j in range(k)]
    return nblocks, bexp, tok, dest, swp.reshape(nsp, 1), pos_list


def kernel(x, topk_weights, topk_ids, w1, w2):
    t, d = x.shape
    n_exp = w1.shape[0]
    k = topk_ids.shape[1]
    ns = t * k
    maxb = n_exp + ns // BT
    nsp = maxb * BT

    nblocks, bexp, tok, dest, swp, pos_list = _routing_metadata(
        topk_weights, topk_ids, n_exp, maxb)
    xs = _sc_permute_rows(x, tok, dest, nsp)
    ys = _tc_grouped_matmul(nblocks, bexp, xs, swp, w1, w2, maxb)
    return _sc_combine(ys, pos_list)


# speedup vs baseline: 1.1543x; 1.1543x over previous
import functools

import jax
import jax.numpy as jnp
from jax import lax
from jax.experimental import pallas as pl
from jax.experimental.pallas import tpu as pltpu
from jax.experimental.pallas import tpu_sc as plsc


BT = 128


def _moe_block_body(nblk_ref, bexp_ref, xs_ref, sw_ref, w1_ref, w2_ref, ys_ref,
                    *, d_ff):
    i = pl.program_id(0)

    @pl.when(i < nblk_ref[0])
    def _():
        xb = xs_ref[...]
        gu = jnp.dot(xb, w1_ref[0], preferred_element_type=jnp.float32)
        g = gu[:, :d_ff]
        u = gu[:, d_ff:]
        h = g * jax.nn.sigmoid(g) * u
        yb = jnp.dot(h, w2_ref[0], preferred_element_type=jnp.float32)
        ys_ref[...] = yb * sw_ref[...]


def _tc_grouped_matmul(nblocks, bexp, xs, sw, w1, w2, maxb):
    nsp, d = xs.shape
    e, _, f2 = w1.shape
    d_ff = w2.shape[1]

    def _clamp(i, nb):
        return jnp.minimum(i, nb[0] - 1)

    grid_spec = pltpu.PrefetchScalarGridSpec(
        num_scalar_prefetch=2,
        grid=(maxb,),
        in_specs=[
            pl.BlockSpec((BT, d), lambda i, nb, be: (_clamp(i, nb), 0)),
            pl.BlockSpec((BT, 1), lambda i, nb, be: (_clamp(i, nb), 0)),
            pl.BlockSpec((1, d, f2), lambda i, nb, be: (be[_clamp(i, nb)], 0, 0)),
            pl.BlockSpec((1, d_ff, d), lambda i, nb, be: (be[_clamp(i, nb)], 0, 0)),
        ],
        out_specs=pl.BlockSpec((BT, d), lambda i, nb, be: (_clamp(i, nb), 0)),
    )
    return pl.pallas_call(
        functools.partial(_moe_block_body, d_ff=d_ff),
        grid_spec=grid_spec,
        out_shape=jax.ShapeDtypeStruct((nsp, d), jnp.float32),
        compiler_params=pltpu.CompilerParams(
            dimension_semantics=("arbitrary",),
        ),
    )(nblocks, bexp, xs, sw, w1, w2)


def _sc_permute_rows(x, tok_sorted, dest, nsp):
    t, d = x.shape
    ns = tok_sorted.shape[0]
    info = plsc.get_sparse_core_info()
    nw = info.num_cores * info.num_subcores
    per_w = ns // nw
    ch = 128
    while per_w % ch:
        ch //= 2
    mesh = plsc.VectorSubcoreMesh(core_axis_name="c", subcore_axis_name="s")

    @functools.partial(
        pl.kernel, mesh=mesh,
        out_type=jax.ShapeDtypeStruct((nsp, d), jnp.float32),
        scratch_types=[
            pltpu.VMEM((ch,), jnp.int32),
            pltpu.VMEM((ch,), jnp.int32),
            pltpu.VMEM((ch, d), jnp.float32),
            pltpu.SemaphoreType.DMA,
        ],
    )
    def k(x_hbm, tok_hbm, dest_hbm, out_hbm, tok_v, dest_v, rows_v, sem):
        wid = lax.axis_index("s") * info.num_cores + lax.axis_index("c")
        base = wid * per_w

        def chunk(c, carry):
            off = base + c * ch
            pltpu.sync_copy(tok_hbm.at[pl.ds(off, ch)], tok_v)
            pltpu.sync_copy(dest_hbm.at[pl.ds(off, ch)], dest_v)
            pltpu.async_copy(x_hbm.at[tok_v], rows_v, sem).wait()
            pltpu.async_copy(rows_v, out_hbm.at[dest_v], sem).wait()
            return carry

        lax.fori_loop(0, per_w // ch, chunk, 0)

    return k(x, tok_sorted, dest)


def _sc_combine(ys, pos_list):
    d = ys.shape[1]
    t = pos_list[0].shape[0]
    info = plsc.get_sparse_core_info()
    nw = info.num_cores * info.num_subcores
    tw = t // nw
    mesh = plsc.VectorSubcoreMesh(core_axis_name="c", subcore_axis_name="s")

    @functools.partial(
        pl.kernel, mesh=mesh,
        out_type=jax.ShapeDtypeStruct((t, d), jnp.float32),
        scratch_types=[
            pltpu.VMEM((tw,), jnp.int32),
            pltpu.VMEM((tw,), jnp.int32),
            pltpu.VMEM((tw, d), jnp.float32),
            pltpu.VMEM((tw, d), jnp.float32),
            pltpu.SemaphoreType.DMA,
            pltpu.SemaphoreType.DMA,
        ],
    )
    def k(ys_hbm, p0_hbm, p1_hbm, out_hbm, i0_v, i1_v, acc_v, tmp_v, s0, s1):
        wid = lax.axis_index("s") * info.num_cores + lax.axis_index("c")
        base = wid * tw

        pltpu.sync_copy(p0_hbm.at[pl.ds(base, tw)], i0_v)
        pltpu.sync_copy(p1_hbm.at[pl.ds(base, tw)], i1_v)
        cp0 = pltpu.async_copy(ys_hbm.at[i0_v], acc_v, s0)
        cp1 = pltpu.async_copy(ys_hbm.at[i1_v], tmp_v, s1)
        cp0.wait()
        cp1.wait()

        def row(r, carry):
            for c in range(d // 16):
                sl = pl.ds(c * 16, 16)
                acc_v[r, sl] = acc_v[r, sl] + tmp_v[r, sl]
            return carry

        lax.fori_loop(0, tw, row, 0)
        pltpu.sync_copy(acc_v, out_hbm.at[pl.ds(base, tw)])

    return k(ys, *pos_list)


def _meta_body(ids_ref, dest_ref, meta_ref, *, n_exp, nb, cs_b):
    ids = ids_ref[...]
    e_iota = lax.broadcasted_iota(jnp.int32, (nb, n_exp, cs_b), 1)
    oh = (ids[:, None, :] == e_iota).astype(jnp.float32)
    c1 = lax.broadcasted_iota(jnp.int32, (cs_b, cs_b), 0)
    c2 = lax.broadcasted_iota(jnp.int32, (cs_b, cs_b), 1)
    tri = (c1 <= c2).astype(jnp.float32)
    intra = lax.dot_general(oh, tri, (((2,), (0,)), ((), ())),
                            preferred_element_type=jnp.float32)
    t1 = jnp.sum(oh, axis=2)
    b1 = lax.broadcasted_iota(jnp.int32, (nb, nb), 0)
    b2 = lax.broadcasted_iota(jnp.int32, (nb, nb), 1)
    tri_s = (b2 < b1).astype(jnp.float32)
    boff = jnp.dot(tri_s, t1, preferred_element_type=jnp.float32)
    csum = intra + boff[:, :, None]
    rank = jnp.sum(oh * csum, axis=1) - 1.0
    counts = jnp.sum(t1, axis=0, keepdims=True)
    nblk = jnp.ceil(counts / BT)
    e1 = lax.broadcasted_iota(jnp.int32, (n_exp, n_exp), 0)
    e2 = lax.broadcasted_iota(jnp.int32, (n_exp, n_exp), 1)
    tri_e = (e1 <= e2).astype(jnp.float32)
    incl = jnp.dot(nblk, tri_e, preferred_element_type=jnp.float32)
    blk_off = incl - nblk
    bo_col = jnp.transpose(blk_off)
    bo_slot = jnp.sum(oh * bo_col[None, :, :], axis=1)
    dest_ref[...] = (bo_slot * BT + rank).astype(jnp.int32)
    nblocks = jnp.sum(nblk)
    incl_col = jnp.transpose(incl)
    j_row = lax.broadcasted_iota(jnp.int32, (1, cs_b), 1).astype(jnp.float32)
    bexp = jnp.sum((incl_col <= j_row).astype(jnp.float32), axis=0,
                   keepdims=True)
    e_row = lax.broadcasted_iota(jnp.int32, (1, n_exp), 1).astype(jnp.float32)
    last_e = jnp.max(jnp.where(counts > 0, e_row, 0.0))
    bexp = jnp.where(j_row < nblocks,
                     jnp.minimum(bexp, float(n_exp - 1)), last_e)
    meta = jnp.where(j_row == cs_b - 1, nblocks, bexp).astype(jnp.int32)
    meta_ref[...] = jnp.broadcast_to(meta, (8, cs_b))


def _routing_metadata(topk_weights, topk_ids, n_exp, maxb):
    t, k = topk_ids.shape
    ns = t * k
    nsp = maxb * BT
    cs_b = 128
    nb = ns // cs_b
    dest2, meta = pl.pallas_call(
        functools.partial(_meta_body, n_exp=n_exp, nb=nb, cs_b=cs_b),
        out_shape=(jax.ShapeDtypeStruct((nb, cs_b), jnp.int32),
                   jax.ShapeDtypeStruct((8, cs_b), jnp.int32)),
    )(topk_ids.reshape(nb, cs_b))
    dest = dest2.reshape(ns)
    nblocks = meta[0, cs_b - 1:cs_b]
    bexp = meta[0, :maxb]
    flat_w = topk_weights.reshape(-1)
    tok = jnp.arange(ns, dtype=jnp.int32) // k
    swp = jnp.zeros((nsp,), jnp.float32).at[dest].set(flat_w)
    pos = dest.reshape(t, k)
    pos_list = [pos[:, j] for j in range(k)]
    return nblocks, bexp, tok, dest, swp.reshape(nsp, 1), pos_list


def kernel(x, topk_weights, topk_ids, w1, w2):
    t, d = x.shape
    n_exp = w1.shape[0]
    k = topk_ids.shape[1]
    ns = t * k
    maxb = n_exp + ns // BT
    nsp = maxb * BT

    nblocks, bexp, tok, dest, swp, pos_list = _routing_metadata(
        topk_weights, topk_ids, n_exp, maxb)
    xs = _sc_permute_rows(x, tok, dest, nsp)
    ys = _tc_grouped_matmul(nblocks, bexp, xs, swp, w1, w2, maxb)
    return _sc_combine(ys, pos_list)
